# Initial kernel scaffold; baseline (speedup 1.0000x reference)
import jax
import jax.numpy as jnp
from jax.experimental import pallas as pl


def kernel(x, edge_index, W1, al1, ar1, b1, W2, al2, ar2, b2, resW2):
    raise NotImplementedError("write your pallas kernel here")


# speedup vs baseline: 15.3199x; 15.3199x over previous
import functools

import jax
import jax.numpy as jnp
from jax import lax
from jax.experimental import pallas as pl
from jax.experimental.pallas import tpu as pltpu
from jax.experimental.pallas import tpu_sc as plsc

N = 10000
E = 320000
IN = 128
H = 8
D = 32
HD = 256
NEG = 0.2

NC = 2
NS = 16
NW = NC * NS
EPW = E // NW
CH = 40
NCHUNK = EPW // CH
NP = 10240
RPT = NP // NS
ZC = 64
L = 16


def _tc_prep1(x, W1, M1):
    def body(x_ref, w_ref, m_ref, fa_ref, fb_ref, att_ref):
        f = jnp.dot(x_ref[...], w_ref[...], preferred_element_type=jnp.float32)
        fa_ref[...] = f[:, :128]
        fb_ref[...] = f[:, 128:]
        att_ref[...] = jnp.dot(f, m_ref[...], preferred_element_type=jnp.float32)

    R = 400
    return pl.pallas_call(
        body,
        grid=(N // R,),
        in_specs=[
            pl.BlockSpec((R, IN), lambda i: (i, 0)),
            pl.BlockSpec((IN, HD), lambda i: (0, 0)),
            pl.BlockSpec((HD, 128), lambda i: (0, 0)),
        ],
        out_specs=[
            pl.BlockSpec((R, 128), lambda i: (i, 0)),
            pl.BlockSpec((R, 128), lambda i: (i, 0)),
            pl.BlockSpec((R, 128), lambda i: (i, 0)),
        ],
        out_shape=[
            jax.ShapeDtypeStruct((N, 128), jnp.float32),
            jax.ShapeDtypeStruct((N, 128), jnp.float32),
            jax.ShapeDtypeStruct((N, 128), jnp.float32),
        ],
    )(x, W1, M1)


def _tc_mid(oA, oB, b1, W2, M2, resW2, b2):
    def body(oa_ref, ob_ref, b1_ref, w2_ref, m2_ref, rw_ref, b2_ref,
             fa_ref, fb_ref, att_ref, res_ref):
        ha = oa_ref[0] + oa_ref[1]
        hb = ob_ref[0] + ob_ref[1]
        h = jnp.concatenate([ha, hb], axis=1) + b1_ref[...]
        f2 = jnp.dot(h, w2_ref[...], preferred_element_type=jnp.float32)
        fa_ref[...] = f2[:, :128]
        fb_ref[...] = f2[:, 128:]
        att_ref[...] = jnp.dot(f2, m2_ref[...], preferred_element_type=jnp.float32)
        res_ref[...] = (jnp.dot(h, rw_ref[...], preferred_element_type=jnp.float32)
                        + b2_ref[...])

    R = 400
    return pl.pallas_call(
        body,
        grid=(N // R,),
        in_specs=[
            pl.BlockSpec((2, R, 128), lambda i: (0, i, 0)),
            pl.BlockSpec((2, R, 128), lambda i: (0, i, 0)),
            pl.BlockSpec((1, HD), lambda i: (0, 0)),
            pl.BlockSpec((HD, HD), lambda i: (0, 0)),
            pl.BlockSpec((HD, 128), lambda i: (0, 0)),
            pl.BlockSpec((HD, HD), lambda i: (0, 0)),
            pl.BlockSpec((1, HD), lambda i: (0, 0)),
        ],
        out_specs=[
            pl.BlockSpec((R, 128), lambda i: (i, 0)),
            pl.BlockSpec((R, 128), lambda i: (i, 0)),
            pl.BlockSpec((R, 128), lambda i: (i, 0)),
            pl.BlockSpec((R, HD), lambda i: (i, 0)),
        ],
        out_shape=[
            jax.ShapeDtypeStruct((N, 128), jnp.float32),
            jax.ShapeDtypeStruct((N, 128), jnp.float32),
            jax.ShapeDtypeStruct((N, 128), jnp.float32),
            jax.ShapeDtypeStruct((N, HD), jnp.float32),
        ],
    )(oA, oB, b1, W2, M2, resW2, b2)


def _tc_dsum(dnp):
    def body(d_ref, out_ref):
        ds = d_ref[0] + d_ref[1]
        out_ref[...] = jnp.concatenate(
            [ds, jnp.zeros((ds.shape[0], 112), jnp.float32)], axis=1)

    R = 512
    return pl.pallas_call(
        body,
        grid=(NP // R,),
        in_specs=[pl.BlockSpec((2, R, 16), lambda i: (0, i, 0))],
        out_specs=pl.BlockSpec((R, 128), lambda i: (i, 0)),
        out_shape=jax.ShapeDtypeStruct((NP, 128), jnp.float32),
    )(dnp)


def _tc_final(oA, oB, res):
    def body(oa_ref, ob_ref, res_ref, out_ref):
        ha = oa_ref[0] + oa_ref[1]
        hb = ob_ref[0] + ob_ref[1]
        out_ref[...] = jnp.concatenate([ha, hb], axis=1) + res_ref[...]

    R = 400
    return pl.pallas_call(
        body,
        grid=(N // R,),
        in_specs=[
            pl.BlockSpec((2, R, 128), lambda i: (0, i, 0)),
            pl.BlockSpec((2, R, 128), lambda i: (0, i, 0)),
            pl.BlockSpec((R, HD), lambda i: (i, 0)),
        ],
        out_specs=pl.BlockSpec((R, HD), lambda i: (i, 0)),
        out_shape=jax.ShapeDtypeStruct((N, HD), jnp.float32),
    )(oA, oB, res)


def _sc_pass1(att, src, dst):
    mesh = plsc.VectorSubcoreMesh(core_axis_name="c", subcore_axis_name="s")

    @functools.partial(
        pl.kernel, mesh=mesh,
        out_type=[
            jax.ShapeDtypeStruct((E, 16), jnp.float32),
            jax.ShapeDtypeStruct((2, NP, 16), jnp.float32),
        ],
        scratch_types=[
            pltpu.VMEM((1, CH), jnp.int32),
            pltpu.VMEM((1, CH), jnp.int32),
            pltpu.VMEM((CH, 128), jnp.float32),
            pltpu.VMEM((CH, 128), jnp.float32),
            pltpu.VMEM((CH, 16), jnp.float32),
            pltpu.VMEM((ZC, 16), jnp.float32),
            pltpu.VMEM_SHARED((NP, 16), jnp.float32),
        ],
    )
    def k(att_h, src_h, dst_h, ex_h, dnp_h, sidx, didx, atts, attd,
          exv, zcp, dsh):
        c = lax.axis_index("c")
        s = lax.axis_index("s")
        wid = s * NC + c

        def zrow(i, _):
            zcp[i, :] = jnp.zeros((L,), jnp.float32)
            return 0
        lax.fori_loop(0, ZC, zrow, 0)

        def zz(q, _):
            pltpu.sync_copy(zcp, dsh.at[pl.ds(s * RPT + q * ZC, ZC)])
            return 0
        lax.fori_loop(0, RPT // ZC, zz, 0)
        plsc.subcore_barrier()

        lanes = lax.iota(jnp.int32, L)
        headmask = lanes < H

        def chunk(kk, _):
            base = pl.multiple_of(wid * EPW + kk * CH, 8)
            pltpu.sync_copy(src_h.at[pl.ds(base, CH)], sidx.at[0])
            pltpu.sync_copy(dst_h.at[pl.ds(base, CH)], didx.at[0])
            pltpu.sync_copy(att_h.at[sidx.at[0]], atts)
            pltpu.sync_copy(att_h.at[didx.at[0]], attd)

            def edge(e, _):
                v = atts[e, pl.ds(0, L)] + attd[e, pl.ds(8, L)]
                v = jnp.where(v >= 0.0, v, NEG * v)
                ex = jnp.where(headmask, jnp.exp(v), 0.0)
                exv[e, :] = ex
                return 0
            lax.fori_loop(0, CH, edge, 0)
            pltpu.sync_copy(exv, ex_h.at[pl.ds(base, CH)])
            pltpu.sync_copy(exv, dsh.at[didx.at[0]], add=True)
            return 0
        lax.fori_loop(0, NCHUNK, chunk, 0)

        plsc.subcore_barrier()

        def cpout(q, _):
            pltpu.sync_copy(dsh.at[pl.ds(s * RPT + q * ZC, ZC)], zcp)
            pltpu.sync_copy(zcp, dnp_h.at[c].at[pl.ds(s * RPT + q * ZC, ZC)])
            return 0
        lax.fori_loop(0, RPT // ZC, cpout, 0)

    return k(att, src, dst)


def _sc_pass2(featA, featB, ex, dns, src, dst):
    mesh = plsc.VectorSubcoreMesh(core_axis_name="c", subcore_axis_name="s")

    @functools.partial(
        pl.kernel, mesh=mesh,
        out_type=[
            jax.ShapeDtypeStruct((2, NP, 128), jnp.float32),
            jax.ShapeDtypeStruct((2, NP, 128), jnp.float32),
            jax.ShapeDtypeStruct((E, 16), jnp.float32),
        ],
        scratch_types=[
            pltpu.VMEM((1, CH), jnp.int32),
            pltpu.VMEM((1, CH), jnp.int32),
            pltpu.VMEM((CH, 128), jnp.float32),
            pltpu.VMEM((CH, 16), jnp.float32),
            pltpu.VMEM((CH, 128), jnp.float32),
            pltpu.VMEM((CH, 16), jnp.float32),
            pltpu.VMEM((ZC, 128), jnp.float32),
            pltpu.VMEM_SHARED((NP, 128), jnp.float32),
        ],
    )
    def k(fa_h, fb_h, ex_h, dns_h, src_h, dst_h, outa_h, outb_h, al_h,
          sidx, didx, fbuf, exv, dbuf, avbuf, zcp, osh):
        c = lax.axis_index("c")
        s = lax.axis_index("s")
        wid = s * NC + c

        def zrow(i, _):
            for j in range(128 // L):
                zcp[i, pl.ds(j * L, L)] = jnp.zeros((L,), jnp.float32)
            return 0

        for half in range(2):
            f_h = fa_h if half == 0 else fb_h
            o_h = outa_h if half == 0 else outb_h

            lax.fori_loop(0, ZC, zrow, 0)

            def zz(q, _):
                pltpu.sync_copy(zcp, osh.at[pl.ds(s * RPT + q * ZC, ZC)])
                return 0
            lax.fori_loop(0, RPT // ZC, zz, 0)
            plsc.subcore_barrier()

            def chunk(kk, _):
                base = pl.multiple_of(wid * EPW + kk * CH, 8)
                pltpu.sync_copy(src_h.at[pl.ds(base, CH)], sidx.at[0])
                pltpu.sync_copy(f_h.at[sidx.at[0]], fbuf)
                pltpu.sync_copy(dst_h.at[pl.ds(base, CH)], didx.at[0])
                if half == 0:
                    pltpu.sync_copy(ex_h.at[pl.ds(base, CH)], exv)
                    pltpu.sync_copy(dns_h.at[didx.at[0]], dbuf)
                else:
                    pltpu.sync_copy(al_h.at[pl.ds(base, CH)], avbuf)

                def edge(e, _):
                    if half == 0:
                        dv = dbuf[e, pl.ds(0, L)] + 1e-9
                        arow = exv[e, :] / dv
                        avbuf[e, :] = arow
                    else:
                        arow = avbuf[e, :]
                    for j in range(4):
                        a = arow[half * 4 + j]
                        fbuf[e, pl.ds(2 * j * L, L)] = fbuf[e, pl.ds(2 * j * L, L)] * a
                        fbuf[e, pl.ds((2 * j + 1) * L, L)] = (
                            fbuf[e, pl.ds((2 * j + 1) * L, L)] * a)
                    return 0
                lax.fori_loop(0, CH, edge, 0)
                if half == 0:
                    pltpu.sync_copy(avbuf, al_h.at[pl.ds(base, CH)])
                pltpu.sync_copy(fbuf, osh.at[didx.at[0]], add=True)
                return 0
            lax.fori_loop(0, NCHUNK, chunk, 0)
            plsc.subcore_barrier()

            def cpout(q, _):
                pltpu.sync_copy(osh.at[pl.ds(s * RPT + q * ZC, ZC)], zcp)
                pltpu.sync_copy(zcp, o_h.at[c].at[pl.ds(s * RPT + q * ZC, ZC)])
                return 0
            lax.fori_loop(0, RPT // ZC, cpout, 0)
            plsc.subcore_barrier()

    return k(featA, featB, ex, dns, src, dst)


def _att_proj(al, ar):
    eye = jnp.eye(H, dtype=jnp.float32)
    Ml = (al[:, :, None] * eye[:, None, :]).reshape(HD, H)
    Mr = (ar[:, :, None] * eye[:, None, :]).reshape(HD, H)
    z = jnp.zeros((HD, 128 - 2 * H), jnp.float32)
    return jnp.concatenate([Ml, Mr, z], axis=1)


def kernel(x, edge_index, W1, al1, ar1, b1, W2, al2, ar2, b2, resW2):
    src = edge_index[0].astype(jnp.int32)
    dst = edge_index[1].astype(jnp.int32)
    M1 = _att_proj(al1, ar1)
    M2 = _att_proj(al2, ar2)

    fA1, fB1, att1 = _tc_prep1(x, W1, M1)
    ex1, dnp1 = _sc_pass1(att1, src, dst)
    dns1 = _tc_dsum(dnp1)
    oA1, oB1, _ = _sc_pass2(fA1, fB1, ex1, dns1, src, dst)

    fA2, fB2, att2, res = _tc_mid(oA1, oB1, b1.reshape(1, HD), W2, M2,
                                  resW2, b2.reshape(1, HD))
    ex2, dnp2 = _sc_pass1(att2, src, dst)
    dns2 = _tc_dsum(dnp2)
    oA2, oB2, _ = _sc_pass2(fA2, fB2, ex2, dns2, src, dst)

    return _tc_final(oA2, oB2, res)


# speedup vs baseline: 17.0292x; 1.1116x over previous
import functools

import jax
import jax.numpy as jnp
from jax import lax
from jax.experimental import pallas as pl
from jax.experimental.pallas import tpu as pltpu
from jax.experimental.pallas import tpu_sc as plsc

N = 10000
E = 320000
IN = 128
H = 8
D = 32
HD = 256
NEG = 0.2

NC = 2
NS = 16
NW = NC * NS
EPW = E // NW
CH = 40
NCHUNK = EPW // CH
CH1 = 80
NCHUNK1 = EPW // CH1
NP = 10240
RPT = NP // NS
ZC = 64
L = 16


def _tc_prep1(x, W1, M1):
    def body(x_ref, w_ref, m_ref, fa_ref, fb_ref, att_ref):
        f = jnp.dot(x_ref[...], w_ref[...], preferred_element_type=jnp.float32)
        fa_ref[...] = f[:, :128]
        fb_ref[...] = f[:, 128:]
        att_ref[...] = jnp.dot(f, m_ref[...], preferred_element_type=jnp.float32)

    R = 400
    return pl.pallas_call(
        body,
        grid=(N // R,),
        in_specs=[
            pl.BlockSpec((R, IN), lambda i: (i, 0)),
            pl.BlockSpec((IN, HD), lambda i: (0, 0)),
            pl.BlockSpec((HD, 128), lambda i: (0, 0)),
        ],
        out_specs=[
            pl.BlockSpec((R, 128), lambda i: (i, 0)),
            pl.BlockSpec((R, 128), lambda i: (i, 0)),
            pl.BlockSpec((R, 128), lambda i: (i, 0)),
        ],
        out_shape=[
            jax.ShapeDtypeStruct((N, 128), jnp.float32),
            jax.ShapeDtypeStruct((N, 128), jnp.float32),
            jax.ShapeDtypeStruct((N, 128), jnp.float32),
        ],
    )(x, W1, M1)


def _tc_mid(oA, oB, b1, W2, M2, resW2, b2):
    def body(oa_ref, ob_ref, b1_ref, w2_ref, m2_ref, rw_ref, b2_ref,
             fa_ref, fb_ref, att_ref, res_ref):
        ha = oa_ref[0] + oa_ref[1]
        hb = ob_ref[0] + ob_ref[1]
        h = jnp.concatenate([ha, hb], axis=1) + b1_ref[...]
        f2 = jnp.dot(h, w2_ref[...], preferred_element_type=jnp.float32)
        fa_ref[...] = f2[:, :128]
        fb_ref[...] = f2[:, 128:]
        att_ref[...] = jnp.dot(f2, m2_ref[...], preferred_element_type=jnp.float32)
        res_ref[...] = (jnp.dot(h, rw_ref[...], preferred_element_type=jnp.float32)
                        + b2_ref[...])

    R = 400
    return pl.pallas_call(
        body,
        grid=(N // R,),
        in_specs=[
            pl.BlockSpec((2, R, 128), lambda i: (0, i, 0)),
            pl.BlockSpec((2, R, 128), lambda i: (0, i, 0)),
            pl.BlockSpec((1, HD), lambda i: (0, 0)),
            pl.BlockSpec((HD, HD), lambda i: (0, 0)),
            pl.BlockSpec((HD, 128), lambda i: (0, 0)),
            pl.BlockSpec((HD, HD), lambda i: (0, 0)),
            pl.BlockSpec((1, HD), lambda i: (0, 0)),
        ],
        out_specs=[
            pl.BlockSpec((R, 128), lambda i: (i, 0)),
            pl.BlockSpec((R, 128), lambda i: (i, 0)),
            pl.BlockSpec((R, 128), lambda i: (i, 0)),
            pl.BlockSpec((R, HD), lambda i: (i, 0)),
        ],
        out_shape=[
            jax.ShapeDtypeStruct((N, 128), jnp.float32),
            jax.ShapeDtypeStruct((N, 128), jnp.float32),
            jax.ShapeDtypeStruct((N, 128), jnp.float32),
            jax.ShapeDtypeStruct((N, HD), jnp.float32),
        ],
    )(oA, oB, b1, W2, M2, resW2, b2)


def _tc_dsum(dnp):
    def body(d_ref, out_ref):
        ds = d_ref[0] + d_ref[1]
        out_ref[...] = jnp.concatenate(
            [ds, jnp.zeros((ds.shape[0], 112), jnp.float32)], axis=1)

    R = 512
    return pl.pallas_call(
        body,
        grid=(NP // R,),
        in_specs=[pl.BlockSpec((2, R, 16), lambda i: (0, i, 0))],
        out_specs=pl.BlockSpec((R, 128), lambda i: (i, 0)),
        out_shape=jax.ShapeDtypeStruct((NP, 128), jnp.float32),
    )(dnp)


def _tc_final(oA, oB, res):
    def body(oa_ref, ob_ref, res_ref, out_ref):
        ha = oa_ref[0] + oa_ref[1]
        hb = ob_ref[0] + ob_ref[1]
        out_ref[...] = jnp.concatenate([ha, hb], axis=1) + res_ref[...]

    R = 400
    return pl.pallas_call(
        body,
        grid=(N // R,),
        in_specs=[
            pl.BlockSpec((2, R, 128), lambda i: (0, i, 0)),
            pl.BlockSpec((2, R, 128), lambda i: (0, i, 0)),
            pl.BlockSpec((R, HD), lambda i: (i, 0)),
        ],
        out_specs=pl.BlockSpec((R, HD), lambda i: (i, 0)),
        out_shape=jax.ShapeDtypeStruct((N, HD), jnp.float32),
    )(oA, oB, res)


def _sc_pass1(att, src, dst):
    mesh = plsc.VectorSubcoreMesh(core_axis_name="c", subcore_axis_name="s")

    @functools.partial(
        pl.kernel, mesh=mesh,
        out_type=[
            jax.ShapeDtypeStruct((E, 16), jnp.float32),
            jax.ShapeDtypeStruct((2, NP, 16), jnp.float32),
        ],
        scratch_types=[
            pltpu.VMEM((1, CH1), jnp.int32),
            pltpu.VMEM((1, CH1), jnp.int32),
            pltpu.VMEM((CH1, 128), jnp.float32),
            pltpu.VMEM((CH1, 128), jnp.float32),
            pltpu.VMEM((CH1, 16), jnp.float32),
            pltpu.VMEM((ZC, 16), jnp.float32),
            pltpu.VMEM_SHARED((NP, 16), jnp.float32),
        ],
    )
    def k(att_h, src_h, dst_h, ex_h, dnp_h, sidx, didx, atts, attd,
          exv, zcp, dsh):
        c = lax.axis_index("c")
        s = lax.axis_index("s")
        wid = s * NC + c

        def zrow(i, _):
            zcp[i, :] = jnp.zeros((L,), jnp.float32)
            return 0
        lax.fori_loop(0, ZC, zrow, 0)

        def zz(q, _):
            pltpu.sync_copy(zcp, dsh.at[pl.ds(s * RPT + q * ZC, ZC)])
            return 0
        lax.fori_loop(0, RPT // ZC, zz, 0)
        plsc.subcore_barrier()

        lanes = lax.iota(jnp.int32, L)
        headmask = lanes < H

        def chunk(kk, _):
            base = pl.multiple_of(wid * EPW + kk * CH1, 8)
            pltpu.sync_copy(src_h.at[pl.ds(base, CH1)], sidx.at[0])
            pltpu.sync_copy(dst_h.at[pl.ds(base, CH1)], didx.at[0])
            pltpu.sync_copy(att_h.at[sidx.at[0]], atts)
            pltpu.sync_copy(att_h.at[didx.at[0]], attd)

            def edge(e, _):
                v = atts[e, pl.ds(0, L)] + attd[e, pl.ds(8, L)]
                v = jnp.where(v >= 0.0, v, NEG * v)
                ex = jnp.where(headmask, jnp.exp(v), 0.0)
                exv[e, :] = ex
                return 0
            lax.fori_loop(0, CH1, edge, 0)
            pltpu.sync_copy(exv, ex_h.at[pl.ds(base, CH1)])
            pltpu.sync_copy(exv, dsh.at[didx.at[0]], add=True)
            return 0
        lax.fori_loop(0, NCHUNK1, chunk, 0)

        plsc.subcore_barrier()

        def cpout(q, _):
            pltpu.sync_copy(dsh.at[pl.ds(s * RPT + q * ZC, ZC)], zcp)
            pltpu.sync_copy(zcp, dnp_h.at[c].at[pl.ds(s * RPT + q * ZC, ZC)])
            return 0
        lax.fori_loop(0, RPT // ZC, cpout, 0)

    return k(att, src, dst)


def _sc_pass2(featA, featB, ex, dns, src, dst):
    mesh = plsc.VectorSubcoreMesh(core_axis_name="c", subcore_axis_name="s")

    @functools.partial(
        pl.kernel, mesh=mesh,
        out_type=[
            jax.ShapeDtypeStruct((2, NP, 128), jnp.float32),
            jax.ShapeDtypeStruct((2, NP, 128), jnp.float32),
            jax.ShapeDtypeStruct((E, 16), jnp.float32),
        ],
        scratch_types=[
            pltpu.VMEM((1, CH), jnp.int32),
            pltpu.VMEM((1, CH), jnp.int32),
            pltpu.VMEM((CH, 128), jnp.float32),
            pltpu.VMEM((CH, 16), jnp.float32),
            pltpu.VMEM((CH, 128), jnp.float32),
            pltpu.VMEM((CH, 16), jnp.float32),
            pltpu.VMEM((ZC, 128), jnp.float32),
            pltpu.VMEM_SHARED((NP, 128), jnp.float32),
        ],
    )
    def k(fa_h, fb_h, ex_h, dns_h, src_h, dst_h, outa_h, outb_h, al_h,
          sidx, didx, fbuf, exv, dbuf, avbuf, zcp, osh):
        c = lax.axis_index("c")
        s = lax.axis_index("s")
        wid = s * NC + c

        def zrow(i, _):
            for j in range(128 // L):
                zcp[i, pl.ds(j * L, L)] = jnp.zeros((L,), jnp.float32)
            return 0

        for half in range(2):
            f_h = fa_h if half == 0 else fb_h
            o_h = outa_h if half == 0 else outb_h

            lax.fori_loop(0, ZC, zrow, 0)

            def zz(q, _):
                pltpu.sync_copy(zcp, osh.at[pl.ds(s * RPT + q * ZC, ZC)])
                return 0
            lax.fori_loop(0, RPT // ZC, zz, 0)
            plsc.subcore_barrier()

            def chunk(kk, _):
                base = pl.multiple_of(wid * EPW + kk * CH, 8)
                pltpu.sync_copy(src_h.at[pl.ds(base, CH)], sidx.at[0])
                pltpu.sync_copy(f_h.at[sidx.at[0]], fbuf)
                pltpu.sync_copy(dst_h.at[pl.ds(base, CH)], didx.at[0])
                if half == 0:
                    pltpu.sync_copy(ex_h.at[pl.ds(base, CH)], exv)
                    pltpu.sync_copy(dns_h.at[didx.at[0]], dbuf)
                else:
                    pltpu.sync_copy(al_h.at[pl.ds(base, CH)], avbuf)

                def edge(e, _):
                    if half == 0:
                        dv = dbuf[e, pl.ds(0, L)] + 1e-9
                        arow = exv[e, :] / dv
                        avbuf[e, :] = arow
                    else:
                        arow = avbuf[e, :]
                    for j in range(4):
                        a = arow[half * 4 + j]
                        fbuf[e, pl.ds(2 * j * L, L)] = fbuf[e, pl.ds(2 * j * L, L)] * a
                        fbuf[e, pl.ds((2 * j + 1) * L, L)] = (
                            fbuf[e, pl.ds((2 * j + 1) * L, L)] * a)
                    return 0
                lax.fori_loop(0, CH, edge, 0)
                if half == 0:
                    pltpu.sync_copy(avbuf, al_h.at[pl.ds(base, CH)])
                pltpu.sync_copy(fbuf, osh.at[didx.at[0]], add=True)
                return 0
            lax.fori_loop(0, NCHUNK, chunk, 0)
            plsc.subcore_barrier()

            def cpout(q, _):
                pltpu.sync_copy(osh.at[pl.ds(s * RPT + q * ZC, ZC)], zcp)
                pltpu.sync_copy(zcp, o_h.at[c].at[pl.ds(s * RPT + q * ZC, ZC)])
                return 0
            lax.fori_loop(0, RPT // ZC, cpout, 0)
            plsc.subcore_barrier()

    return k(featA, featB, ex, dns, src, dst)


def _att_proj(al, ar):
    eye = jnp.eye(H, dtype=jnp.float32)
    Ml = (al[:, :, None] * eye[:, None, :]).reshape(HD, H)
    Mr = (ar[:, :, None] * eye[:, None, :]).reshape(HD, H)
    z = jnp.zeros((HD, 128 - 2 * H), jnp.float32)
    return jnp.concatenate([Ml, Mr, z], axis=1)


def kernel(x, edge_index, W1, al1, ar1, b1, W2, al2, ar2, b2, resW2):
    src = edge_index[0].astype(jnp.int32)
    dst = edge_index[1].astype(jnp.int32)
    M1 = _att_proj(al1, ar1)
    M2 = _att_proj(al2, ar2)

    fA1, fB1, att1 = _tc_prep1(x, W1, M1)
    ex1, dnp1 = _sc_pass1(att1, src, dst)
    dns1 = _tc_dsum(dnp1)
    oA1, oB1, _ = _sc_pass2(fA1, fB1, ex1, dns1, src, dst)

    fA2, fB2, att2, res = _tc_mid(oA1, oB1, b1.reshape(1, HD), W2, M2,
                                  resW2, b2.reshape(1, HD))
    ex2, dnp2 = _sc_pass1(att2, src, dst)
    dns2 = _tc_dsum(dnp2)
    oA2, oB2, _ = _sc_pass2(fA2, fB2, ex2, dns2, src, dst)

    return _tc_final(oA2, oB2, res)


# speedup vs baseline: 23.0735x; 1.3549x over previous
import functools

import jax
import jax.numpy as jnp
from jax import lax
from jax.experimental import pallas as pl
from jax.experimental.pallas import tpu as pltpu
from jax.experimental.pallas import tpu_sc as plsc

N = 10000
E = 320000
IN = 128
H = 8
D = 32
HD = 256
NEG = 0.2

NC = 2
NS = 16
NW = NC * NS
EPW = E // NW
CH = 40
NCHUNK = EPW // CH
CH1 = 80
NCHUNK1 = EPW // CH1
NP = 10240
RPT = NP // NS
ZC = 64
L = 16


def _tc_prep1(x, W1, M1):
    def body(x_ref, w_ref, m_ref, fa_ref, fb_ref, att_ref):
        f = jnp.dot(x_ref[...], w_ref[...], preferred_element_type=jnp.float32)
        fa_ref[...] = f[:, :128]
        fb_ref[...] = f[:, 128:]
        att_ref[...] = jnp.dot(f, m_ref[...], preferred_element_type=jnp.float32)

    R = 400
    return pl.pallas_call(
        body,
        grid=(N // R,),
        in_specs=[
            pl.BlockSpec((R, IN), lambda i: (i, 0)),
            pl.BlockSpec((IN, HD), lambda i: (0, 0)),
            pl.BlockSpec((HD, 128), lambda i: (0, 0)),
        ],
        out_specs=[
            pl.BlockSpec((R, 128), lambda i: (i, 0)),
            pl.BlockSpec((R, 128), lambda i: (i, 0)),
            pl.BlockSpec((R, 128), lambda i: (i, 0)),
        ],
        out_shape=[
            jax.ShapeDtypeStruct((N, 128), jnp.float32),
            jax.ShapeDtypeStruct((N, 128), jnp.float32),
            jax.ShapeDtypeStruct((N, 128), jnp.float32),
        ],
    )(x, W1, M1)


def _tc_mid(oA, oB, b1, W2, M2, resW2, b2):
    def body(oa_ref, ob_ref, b1_ref, w2_ref, m2_ref, rw_ref, b2_ref,
             fa_ref, fb_ref, att_ref, res_ref):
        ha = oa_ref[0] + oa_ref[1]
        hb = ob_ref[0] + ob_ref[1]
        h = jnp.concatenate([ha, hb], axis=1) + b1_ref[...]
        f2 = jnp.dot(h, w2_ref[...], preferred_element_type=jnp.float32)
        fa_ref[...] = f2[:, :128]
        fb_ref[...] = f2[:, 128:]
        att_ref[...] = jnp.dot(f2, m2_ref[...], preferred_element_type=jnp.float32)
        res_ref[...] = (jnp.dot(h, rw_ref[...], preferred_element_type=jnp.float32)
                        + b2_ref[...])

    R = 400
    return pl.pallas_call(
        body,
        grid=(N // R,),
        in_specs=[
            pl.BlockSpec((2, R, 128), lambda i: (0, i, 0)),
            pl.BlockSpec((2, R, 128), lambda i: (0, i, 0)),
            pl.BlockSpec((1, HD), lambda i: (0, 0)),
            pl.BlockSpec((HD, HD), lambda i: (0, 0)),
            pl.BlockSpec((HD, 128), lambda i: (0, 0)),
            pl.BlockSpec((HD, HD), lambda i: (0, 0)),
            pl.BlockSpec((1, HD), lambda i: (0, 0)),
        ],
        out_specs=[
            pl.BlockSpec((R, 128), lambda i: (i, 0)),
            pl.BlockSpec((R, 128), lambda i: (i, 0)),
            pl.BlockSpec((R, 128), lambda i: (i, 0)),
            pl.BlockSpec((R, HD), lambda i: (i, 0)),
        ],
        out_shape=[
            jax.ShapeDtypeStruct((N, 128), jnp.float32),
            jax.ShapeDtypeStruct((N, 128), jnp.float32),
            jax.ShapeDtypeStruct((N, 128), jnp.float32),
            jax.ShapeDtypeStruct((N, HD), jnp.float32),
        ],
    )(oA, oB, b1, W2, M2, resW2, b2)


def _tc_dsum(dnp):
    def body(d_ref, out_ref):
        ds = d_ref[0] + d_ref[1]
        out_ref[...] = jnp.concatenate(
            [ds, jnp.zeros((ds.shape[0], 112), jnp.float32)], axis=1)

    R = 512
    return pl.pallas_call(
        body,
        grid=(NP // R,),
        in_specs=[pl.BlockSpec((2, R, 16), lambda i: (0, i, 0))],
        out_specs=pl.BlockSpec((R, 128), lambda i: (i, 0)),
        out_shape=jax.ShapeDtypeStruct((NP, 128), jnp.float32),
    )(dnp)


def _tc_final(oA, oB, res):
    def body(oa_ref, ob_ref, res_ref, out_ref):
        ha = oa_ref[0] + oa_ref[1]
        hb = ob_ref[0] + ob_ref[1]
        out_ref[...] = jnp.concatenate([ha, hb], axis=1) + res_ref[...]

    R = 400
    return pl.pallas_call(
        body,
        grid=(N // R,),
        in_specs=[
            pl.BlockSpec((2, R, 128), lambda i: (0, i, 0)),
            pl.BlockSpec((2, R, 128), lambda i: (0, i, 0)),
            pl.BlockSpec((R, HD), lambda i: (i, 0)),
        ],
        out_specs=pl.BlockSpec((R, HD), lambda i: (i, 0)),
        out_shape=jax.ShapeDtypeStruct((N, HD), jnp.float32),
    )(oA, oB, res)


def _sc_pass1(att, src, dst):
    mesh = plsc.VectorSubcoreMesh(core_axis_name="c", subcore_axis_name="s")

    @functools.partial(
        pl.kernel, mesh=mesh,
        out_type=[
            jax.ShapeDtypeStruct((E, 16), jnp.float32),
            jax.ShapeDtypeStruct((2, NP, 16), jnp.float32),
        ],
        scratch_types=[
            pltpu.VMEM((1, CH1), jnp.int32),
            pltpu.VMEM((1, CH1), jnp.int32),
            pltpu.VMEM((CH1, 128), jnp.float32),
            pltpu.VMEM((CH1, 128), jnp.float32),
            pltpu.VMEM((CH1, 16), jnp.float32),
            pltpu.VMEM((ZC, 16), jnp.float32),
            pltpu.VMEM_SHARED((NP, 16), jnp.float32),
            pltpu.SemaphoreType.DMA,
            pltpu.SemaphoreType.DMA,
        ],
    )
    def k(att_h, src_h, dst_h, ex_h, dnp_h, sidx, didx, atts, attd,
          exv, zcp, dsh, sem1, sem2):
        c = lax.axis_index("c")
        s = lax.axis_index("s")
        wid = s * NC + c

        def zrow(i, _):
            zcp[i, :] = jnp.zeros((L,), jnp.float32)
            return 0
        lax.fori_loop(0, ZC, zrow, 0)

        def zz(q, _):
            pltpu.sync_copy(zcp, dsh.at[pl.ds(s * RPT + q * ZC, ZC)])
            return 0
        lax.fori_loop(0, RPT // ZC, zz, 0)
        plsc.subcore_barrier()

        lanes = lax.iota(jnp.int32, L)
        headmask = lanes < H

        def chunk(kk, _):
            base = pl.multiple_of(wid * EPW + kk * CH1, 8)
            pltpu.sync_copy(src_h.at[pl.ds(base, CH1)], sidx.at[0])
            pltpu.sync_copy(dst_h.at[pl.ds(base, CH1)], didx.at[0])
            ca = pltpu.async_copy(att_h.at[sidx.at[0]], atts, sem1)
            cb = pltpu.async_copy(att_h.at[didx.at[0]], attd, sem2)
            ca.wait()
            cb.wait()

            def edge(e, _):
                v = atts[e, pl.ds(0, L)] + attd[e, pl.ds(8, L)]
                v = jnp.where(v >= 0.0, v, NEG * v)
                ex = jnp.where(headmask, jnp.exp(v), 0.0)
                exv[e, :] = ex
                return 0
            lax.fori_loop(0, CH1, edge, 0)
            pltpu.sync_copy(exv, ex_h.at[pl.ds(base, CH1)])
            pltpu.sync_copy(exv, dsh.at[didx.at[0]], add=True)
            return 0
        lax.fori_loop(0, NCHUNK1, chunk, 0)

        plsc.subcore_barrier()

        def cpout(q, _):
            pltpu.sync_copy(dsh.at[pl.ds(s * RPT + q * ZC, ZC)], zcp)
            pltpu.sync_copy(zcp, dnp_h.at[c].at[pl.ds(s * RPT + q * ZC, ZC)])
            return 0
        lax.fori_loop(0, RPT // ZC, cpout, 0)

    return k(att, src, dst)


def _sc_pass2(featA, featB, ex, dns, src, dst):
    mesh = plsc.VectorSubcoreMesh(core_axis_name="c", subcore_axis_name="s")

    @functools.partial(
        pl.kernel, mesh=mesh,
        out_type=[
            jax.ShapeDtypeStruct((2, NP, 128), jnp.float32),
            jax.ShapeDtypeStruct((2, NP, 128), jnp.float32),
            jax.ShapeDtypeStruct((E, 16), jnp.float32),
        ],
        scratch_types=[
            pltpu.VMEM((1, CH), jnp.int32),
            pltpu.VMEM((1, CH), jnp.int32),
            pltpu.VMEM((CH, 128), jnp.float32),
            pltpu.VMEM((CH, 16), jnp.float32),
            pltpu.VMEM((CH, 128), jnp.float32),
            pltpu.VMEM((CH, 16), jnp.float32),
            pltpu.VMEM((ZC, 128), jnp.float32),
            pltpu.VMEM_SHARED((NP, 128), jnp.float32),
            pltpu.SemaphoreType.DMA,
            pltpu.SemaphoreType.DMA,
            pltpu.SemaphoreType.DMA,
        ],
    )
    def k(fa_h, fb_h, ex_h, dns_h, src_h, dst_h, outa_h, outb_h, al_h,
          sidx, didx, fbuf, exv, dbuf, avbuf, zcp, osh, sem1, sem2, sem3):
        c = lax.axis_index("c")
        s = lax.axis_index("s")
        wid = s * NC + c

        def zrow(i, _):
            for j in range(128 // L):
                zcp[i, pl.ds(j * L, L)] = jnp.zeros((L,), jnp.float32)
            return 0

        for half in range(2):
            f_h = fa_h if half == 0 else fb_h
            o_h = outa_h if half == 0 else outb_h

            lax.fori_loop(0, ZC, zrow, 0)

            def zz(q, _):
                pltpu.sync_copy(zcp, osh.at[pl.ds(s * RPT + q * ZC, ZC)])
                return 0
            lax.fori_loop(0, RPT // ZC, zz, 0)
            plsc.subcore_barrier()

            def chunk(kk, _):
                base = pl.multiple_of(wid * EPW + kk * CH, 8)
                pltpu.sync_copy(src_h.at[pl.ds(base, CH)], sidx.at[0])
                pltpu.sync_copy(dst_h.at[pl.ds(base, CH)], didx.at[0])
                c1 = pltpu.async_copy(f_h.at[sidx.at[0]], fbuf, sem1)
                if half == 0:
                    c2 = pltpu.async_copy(dns_h.at[didx.at[0]], dbuf, sem2)
                    c3 = pltpu.async_copy(ex_h.at[pl.ds(base, CH)], exv, sem3)
                else:
                    c2 = pltpu.async_copy(al_h.at[pl.ds(base, CH)], avbuf, sem2)
                c1.wait()
                c2.wait()
                if half == 0:
                    c3.wait()

                def edge(e, _):
                    if half == 0:
                        dv = dbuf[e, pl.ds(0, L)] + 1e-9
                        arow = exv[e, :] / dv
                        avbuf[e, :] = arow
                    else:
                        arow = avbuf[e, :]
                    for j in range(4):
                        a = arow[half * 4 + j]
                        fbuf[e, pl.ds(2 * j * L, L)] = fbuf[e, pl.ds(2 * j * L, L)] * a
                        fbuf[e, pl.ds((2 * j + 1) * L, L)] = (
                            fbuf[e, pl.ds((2 * j + 1) * L, L)] * a)
                    return 0
                lax.fori_loop(0, CH, edge, 0)
                if half == 0:
                    pltpu.sync_copy(avbuf, al_h.at[pl.ds(base, CH)])
                pltpu.sync_copy(fbuf, osh.at[didx.at[0]], add=True)
                return 0
            lax.fori_loop(0, NCHUNK, chunk, 0)
            plsc.subcore_barrier()

            def cpout(q, _):
                pltpu.sync_copy(osh.at[pl.ds(s * RPT + q * ZC, ZC)], zcp)
                pltpu.sync_copy(zcp, o_h.at[c].at[pl.ds(s * RPT + q * ZC, ZC)])
                return 0
            lax.fori_loop(0, RPT // ZC, cpout, 0)
            plsc.subcore_barrier()

    return k(featA, featB, ex, dns, src, dst)


def _att_proj(al, ar):
    eye = jnp.eye(H, dtype=jnp.float32)
    Ml = (al[:, :, None] * eye[:, None, :]).reshape(HD, H)
    Mr = (ar[:, :, None] * eye[:, None, :]).reshape(HD, H)
    z = jnp.zeros((HD, 128 - 2 * H), jnp.float32)
    return jnp.concatenate([Ml, Mr, z], axis=1)


def kernel(x, edge_index, W1, al1, ar1, b1, W2, al2, ar2, b2, resW2):
    src = edge_index[0].astype(jnp.int32)
    dst = edge_index[1].astype(jnp.int32)
    M1 = _att_proj(al1, ar1)
    M2 = _att_proj(al2, ar2)

    fA1, fB1, att1 = _tc_prep1(x, W1, M1)
    ex1, dnp1 = _sc_pass1(att1, src, dst)
    dns1 = _tc_dsum(dnp1)
    oA1, oB1, _ = _sc_pass2(fA1, fB1, ex1, dns1, src, dst)

    fA2, fB2, att2, res = _tc_mid(oA1, oB1, b1.reshape(1, HD), W2, M2,
                                  resW2, b2.reshape(1, HD))
    ex2, dnp2 = _sc_pass1(att2, src, dst)
    dns2 = _tc_dsum(dnp2)
    oA2, oB2, _ = _sc_pass2(fA2, fB2, ex2, dns2, src, dst)

    return _tc_final(oA2, oB2, res)
